# baseline (speedup 1.0000x reference)
import functools

import jax
import jax.numpy as jnp
from jax.experimental import pallas as pl
from jax.experimental.pallas import tpu as pltpu

N = 8192
ROW_BLK = 256
COL_CHUNK = 128
N_ROW_BLKS = N // ROW_BLK
N_COL_CHUNKS = N // COL_CHUNK


def _chamfer_body(a_ref, b_ref, out_ref, colacc_ref, rowsum_ref):
    c = pl.program_id(0)
    r = pl.program_id(1)

    @pl.when((c == 0) & (r == 0))
    def _init():
        rowsum_ref[0] = 0.0

    a = a_ref[0]
    first_row_blk = r == 0

    def chunk_step(k, rowacc):
        bc = b_ref[0, pl.ds(k, 1), :]
        d = jnp.abs(a - bc)
        rowacc = jnp.minimum(rowacc, d)
        colpart = jnp.min(d, axis=0, keepdims=True)
        row = c * N_COL_CHUNKS + k
        old = colacc_ref[pl.ds(row, 1), :]
        colacc_ref[pl.ds(row, 1), :] = jnp.where(
            first_row_blk, colpart, jnp.minimum(old, colpart))
        return rowacc

    rowacc0 = jnp.full((ROW_BLK, COL_CHUNK), jnp.inf, dtype=jnp.float32)
    rowacc = jax.lax.fori_loop(0, N_COL_CHUNKS, chunk_step, rowacc0)
    rowmin = jnp.min(rowacc, axis=1)
    rowsum_ref[0] += jnp.sum(rowmin)

    @pl.when((c == 2) & (r == N_ROW_BLKS - 1))
    def _finish():
        colsum = jnp.sum(colacc_ref[...])
        out_ref[0, 0] = (rowsum_ref[0] + colsum) / N


def _chamfer_call(a3, b3, interpret=False):
    grid = (3, N_ROW_BLKS)
    return pl.pallas_call(
        _chamfer_body,
        grid=grid,
        in_specs=[
            pl.BlockSpec((1, ROW_BLK, 1), lambda c, r: (c, r, 0)),
            pl.BlockSpec((1, N_COL_CHUNKS, COL_CHUNK), lambda c, r: (c, 0, 0)),
        ],
        out_specs=pl.BlockSpec(
            (1, 1), lambda c, r: (0, 0), memory_space=pltpu.SMEM),
        out_shape=jax.ShapeDtypeStruct((1, 1), jnp.float32),
        scratch_shapes=[
            pltpu.VMEM((3 * N_COL_CHUNKS, COL_CHUNK), jnp.float32),
            pltpu.SMEM((1,), jnp.float32),
        ],
        compiler_params=pltpu.CompilerParams(
            dimension_semantics=("arbitrary", "arbitrary")),
        interpret=interpret,
    )(a3, b3)


@jax.jit
def kernel(pred, target):
    a3 = pred.T[:, :, None]
    b3 = target.T.reshape(3, N_COL_CHUNKS, COL_CHUNK)
    out = _chamfer_call(a3, b3)
    return out[0, 0]


# speedup vs baseline: 3.5791x; 3.5791x over previous
import jax
import jax.numpy as jnp
from jax.experimental import pallas as pl
from jax.experimental.pallas import tpu as pltpu

N = 8192
ROW_BLK = 128
COL_CHUNK = 128
N_ROW_BLKS = N // ROW_BLK
N_COL_CHUNKS = N // COL_CHUNK


def _chamfer_body(a_ref, b_ref, out_ref, colacc_ref, rowsum_ref):
    c = pl.program_id(0)
    r = pl.program_id(1)

    @pl.when((c == 0) & (r == 0))
    def _init():
        rowsum_ref[0] = 0.0
        colacc_ref[...] = jnp.full(
            (3 * N_COL_CHUNKS, 8, COL_CHUNK), jnp.inf, dtype=jnp.float32)

    a = jnp.broadcast_to(a_ref[0], (ROW_BLK, COL_CHUNK))

    def chunk_step(k, rowacc):
        bc = b_ref[0, pl.ds(k, 1), :]
        d = jnp.abs(a - bc)
        rowacc = jnp.minimum(rowacc, d)
        colpart = jnp.min(
            d.reshape(ROW_BLK // 8, 8, COL_CHUNK), axis=0)
        row = c * N_COL_CHUNKS + k
        colacc_ref[pl.ds(row, 1), :, :] = jnp.minimum(
            colacc_ref[pl.ds(row, 1), :, :], colpart[None])
        return rowacc

    rowacc0 = jnp.full((ROW_BLK, COL_CHUNK), jnp.inf, dtype=jnp.float32)
    rowacc = jax.lax.fori_loop(0, N_COL_CHUNKS, chunk_step, rowacc0,
                               unroll=4)
    rowmin = jnp.min(rowacc, axis=1)
    rowsum_ref[0] += jnp.sum(rowmin)

    @pl.when((c == 2) & (r == N_ROW_BLKS - 1))
    def _finish():
        colmin = jnp.min(colacc_ref[...], axis=1)
        out_ref[0, 0] = (rowsum_ref[0] + jnp.sum(colmin)) / N


def _chamfer_call(a3, b3, interpret=False):
    grid = (3, N_ROW_BLKS)
    return pl.pallas_call(
        _chamfer_body,
        grid=grid,
        in_specs=[
            pl.BlockSpec((1, ROW_BLK, 1), lambda c, r: (c, r, 0)),
            pl.BlockSpec((1, N_COL_CHUNKS, COL_CHUNK), lambda c, r: (c, 0, 0)),
        ],
        out_specs=pl.BlockSpec(
            (1, 1), lambda c, r: (0, 0), memory_space=pltpu.SMEM),
        out_shape=jax.ShapeDtypeStruct((1, 1), jnp.float32),
        scratch_shapes=[
            pltpu.VMEM((3 * N_COL_CHUNKS, 8, COL_CHUNK), jnp.float32),
            pltpu.SMEM((1,), jnp.float32),
        ],
        compiler_params=pltpu.CompilerParams(
            dimension_semantics=("arbitrary", "arbitrary")),
        interpret=interpret,
    )(a3, b3)


@jax.jit
def kernel(pred, target):
    a3 = pred.T[:, :, None]
    b3 = target.T.reshape(3, N_COL_CHUNKS, COL_CHUNK)
    out = _chamfer_call(a3, b3)
    return out[0, 0]
